# baseline (speedup 1.0000x reference)
import jax
import jax.numpy as jnp
from jax import lax
from jax.experimental import pallas as pl
from jax.experimental.pallas import tpu as pltpu
from jax.experimental.pallas import tpu_sc as plsc

N = 10000
G = 16
NC = 2
NS = 16
NW = NC * NS
L = 16
CH = 128
PCH = 32
BLK = 8
E_PAD = 344064
E_DEG = 344064
NP_ = 10240
RPT = NP_ // NS
ZR = 128


def _deg_body(col_hbm, ew_hbm, out_hbm, acc,
              cv0, cv1, cv2, cv3, ev0, ev1, ev2, st0, st1, st2, zrow,
              se0, se1, se2, sc0, sc1, sc2, sc3, ss0, ss1, ss2):
    cid = lax.axis_index("c")
    sid = lax.axis_index("s")
    e_per_tile = E_DEG // NW
    nchunks = e_per_tile // CH
    base = (cid * NS + sid) * e_per_tile
    zero = jnp.zeros((L,), jnp.float32)
    colv = (cv0, cv1, cv2, cv3)
    ewv = (ev0, ev1, ev2)
    stage = (st0, st1, st2)
    se = (se0, se1, se2)
    sc = (sc0, sc1, sc2, sc3)
    ss = (ss0, ss1, ss2)

    @pl.loop(0, ZR)
    def _(i):
        zrow[i, :] = zero

    @pl.loop(0, RPT // ZR)
    def _(k):
        pltpu.sync_copy(zrow, acc.at[pl.ds(sid * RPT + k * ZR, ZR)])
    plsc.subcore_barrier()

    for g in range(2):
        pltpu.sync_copy(ew_hbm.at[pl.ds(base + g * CH, CH)], ewv[g])
        pltpu.sync_copy(col_hbm.at[pl.ds(base + g * CH, CH)], colv[g])

    def chunk(g, k, kc):
        @pl.when(g >= 2)
        def _():
            pltpu.make_async_copy(stage[(k + 1) % 3], acc.at[colv[0]],
                                  ss[(k + 1) % 3]).wait()

        @pl.when(g + 2 < nchunks)
        def _():
            e2 = base + (g + 2) * CH
            pltpu.async_copy(ew_hbm.at[pl.ds(e2, CH)], ewv[(k + 2) % 3],
                             se[(k + 2) % 3])
            pltpu.async_copy(col_hbm.at[pl.ds(e2, CH)], colv[(kc + 2) % 4],
                             sc[(kc + 2) % 4])

        @pl.when(g >= 2)
        def _():
            pltpu.make_async_copy(ew_hbm.at[pl.ds(base, CH)], ewv[k],
                                  se[k]).wait()
            pltpu.make_async_copy(col_hbm.at[pl.ds(base, CH)], colv[kc],
                                  sc[kc]).wait()

        @pl.loop(0, CH // L)
        def _(kk):
            vew = ewv[k][pl.ds(kk * L, L)]
            for j in range(L):
                stage[k][kk * L + j, :] = jnp.broadcast_to(vew[j], (L,))

        pltpu.async_copy(stage[k], acc.at[colv[kc]], ss[k], add=True)

    nun = 12
    @pl.loop(0, nchunks // nun)
    def _(gg):
        g = gg * nun
        for u in range(nun):
            chunk(g + u, u % 3, u % 4)

    for gl in (nchunks - 2, nchunks - 1):
        pltpu.make_async_copy(stage[gl % 3], acc.at[colv[0]],
                              ss[gl % 3]).wait()

    plsc.subcore_barrier()
    pltpu.sync_copy(acc.at[pl.ds(sid * RPT, RPT)],
                    out_hbm.at[pl.ds(cid * NP_ + sid * RPT, RPT)])


def _make_pass_body(D, SC, DACC):
    nchunks = E_PAD // NW // PCH

    def body(hs_hbm, row2_hbm, col2_hbm, ew2_hbm, out_hbm,
             acc, rb0, rb1, cb0, cb1, eb0, eb1, g0, g1, g2, g3, *rest):
        if DACC == D:
            (sb0, sb1, sg0, sg1, sg2, sg3, ss0, ss1, ss2, ss3) = rest
            cmpb = None
        else:
            (c0, c1, c2, c3,
             sb0, sb1, sg0, sg1, sg2, sg3, ss0, ss1, ss2, ss3) = rest
            cmpb = (c0, c1, c2, c3)
        cid = lax.axis_index("c")
        sid = lax.axis_index("s")
        tid = cid * NS + sid
        cbase = pl.multiple_of(tid * nchunks, 8)
        zero = jnp.zeros((L,), jnp.float32)
        rowb = (rb0, rb1)
        colb = (cb0, cb1)
        ewb = (eb0, eb1)
        gath = (g0, g1, g2, g3)
        cmp_ = gath if DACC == D else cmpb
        sb = (sb0, sb1)
        sg = (sg0, sg1, sg2, sg3)
        ss = (ss0, ss1, ss2, ss3)

        @pl.loop(0, PCH)
        def _(i):
            for d in range(DACC // L):
                cmp_[0][i, pl.ds(d * L, L)] = zero

        @pl.loop(0, RPT // PCH)
        def _(k):
            pltpu.sync_copy(cmp_[0], acc.at[pl.ds(sid * RPT + k * PCH, PCH)])
        plsc.subcore_barrier()

        pltpu.sync_copy(row2_hbm.at[pl.ds(cbase, BLK)], rowb[0])
        pltpu.sync_copy(col2_hbm.at[pl.ds(cbase, BLK)], colb[0])
        pltpu.sync_copy(ew2_hbm.at[pl.ds(cbase, BLK)], ewb[0])
        pltpu.async_copy(hs_hbm.at[rowb[0].at[0]], g0, sg0)
        pltpu.async_copy(hs_hbm.at[rowb[0].at[1]], g1, sg1)

        def scale(k, p, j):
            @pl.loop(0, PCH // L)
            def _(kk):
                vew = ewb[p][j, pl.ds(kk * L, L)]
                for jj in range(L):
                    sval = jnp.broadcast_to(vew[jj], (L,))
                    r = kk * L + jj
                    for d in range(SC // L):
                        cmp_[k][r, pl.ds(d * L, L)] = (
                            gath[k][r, pl.ds(d * L, L)] * sval)

        def chunk(g, u):
            k = u % 4
            k2 = (k + 2) % 4
            p = u // 8
            j = u % 8
            up = u + 2
            p2, j2 = ((up // 8) % 2, up % 8)

            @pl.when(g >= 2)
            def _():
                pltpu.make_async_copy(cmp_[k2], acc.at[colb[0].at[0]],
                                      ss[k2]).wait()

            if u % 8 == 2:
                @pl.when(g + 6 < nchunks)
                def _():
                    cb = pl.multiple_of(cbase + g + 6, 8)
                    pltpu.async_copy(row2_hbm.at[pl.ds(cb, BLK)],
                                     rowb[1 - p], sb[1 - p])
                    pltpu.async_copy(col2_hbm.at[pl.ds(cb, BLK)],
                                     colb[1 - p], sb[1 - p])
                    pltpu.async_copy(ew2_hbm.at[pl.ds(cb, BLK)],
                                     ewb[1 - p], sb[1 - p])

            @pl.when(g + 2 < nchunks)
            def _():
                if up % 8 == 0:
                    pltpu.make_async_copy(row2_hbm.at[pl.ds(cbase, BLK)],
                                          rowb[p2], sb[p2]).wait()
                    pltpu.make_async_copy(col2_hbm.at[pl.ds(cbase, BLK)],
                                          colb[p2], sb[p2]).wait()
                    pltpu.make_async_copy(ew2_hbm.at[pl.ds(cbase, BLK)],
                                          ewb[p2], sb[p2]).wait()
                pltpu.async_copy(hs_hbm.at[rowb[p2].at[j2]], gath[k2],
                                 sg[k2])

            pltpu.make_async_copy(hs_hbm.at[rowb[p].at[j]], gath[k],
                                  sg[k]).wait()
            scale(k, p, j)
            pltpu.async_copy(cmp_[k], acc.at[colb[p].at[j]], ss[k], add=True)

        @pl.loop(0, nchunks // 16)
        def _(gg):
            g = gg * 16
            for u in range(16):
                chunk(g + u, u)

        for gl in (nchunks - 2, nchunks - 1):
            pltpu.make_async_copy(cmp_[gl % 4], acc.at[colb[0].at[0]],
                                  ss[gl % 4]).wait()

        plsc.subcore_barrier()
        pltpu.sync_copy(acc.at[pl.ds(sid * RPT, RPT)],
                        out_hbm.at[pl.ds(cid * NP_ + sid * RPT, RPT)])

    return body


_mesh = plsc.VectorSubcoreMesh(core_axis_name="c", subcore_axis_name="s")

_deg = pl.kernel(
    _deg_body,
    out_type=jax.ShapeDtypeStruct((NC * NP_, L), jnp.float32),
    mesh=_mesh,
    scratch_types=(
        [pltpu.VMEM_SHARED((NP_, L), jnp.float32)]
        + [pltpu.VMEM((CH,), jnp.int32) for _ in range(4)]
        + [pltpu.VMEM((CH,), jnp.float32) for _ in range(3)]
        + [pltpu.VMEM((CH, L), jnp.float32) for _ in range(3)]
        + [pltpu.VMEM((ZR, L), jnp.float32)]
        + [pltpu.SemaphoreType.DMA for _ in range(10)]
    ),
)


def _make_pass(D, SC, DACC):
    extra = ([] if DACC == D
             else [pltpu.VMEM((PCH, DACC), jnp.float32) for _ in range(4)])
    return pl.kernel(
        _make_pass_body(D, SC, DACC),
        out_type=jax.ShapeDtypeStruct((NC * NP_, DACC), jnp.float32),
        mesh=_mesh,
        scratch_types=(
            [pltpu.VMEM_SHARED((NP_, DACC), jnp.float32)]
            + [pltpu.VMEM((BLK, PCH), jnp.int32) for _ in range(4)]
            + [pltpu.VMEM((BLK, PCH), jnp.float32) for _ in range(2)]
            + [pltpu.VMEM((PCH, D), jnp.float32) for _ in range(4)]
            + extra
            + [pltpu.SemaphoreType.DMA for _ in range(10)]
        ),
    )


_pass1 = _make_pass(128, 128, 128)
_pass2 = _make_pass(128, 64, 64)


def _tc1_body(deg_ref, x_ref, w1_ref, hs_ref, dinv_ref):
    deg = deg_ref[0:N, 0:1] + deg_ref[NP_:NP_ + N, 0:1]
    dinv = lax.rsqrt(deg)
    dinv_ref[...] = dinv
    hs_ref[...] = jnp.dot(x_ref[...], w1_ref[...],
                          preferred_element_type=jnp.float32) * dinv


_tc1 = pl.pallas_call(
    _tc1_body,
    out_shape=(jax.ShapeDtypeStruct((N, 128), jnp.float32),
               jax.ShapeDtypeStruct((N, 1), jnp.float32)),
)


def _tc2_body(accp_ref, dinv_ref, b1_ref, w2_ref, hs2_ref):
    dinv = dinv_ref[...]
    h = (accp_ref[0:N] + accp_ref[NP_:NP_ + N]) * dinv + b1_ref[...]
    h = jnp.maximum(h, 0.0)
    hs2_ref[...] = jnp.dot(h, w2_ref[...],
                           preferred_element_type=jnp.float32) * dinv


_tc2 = pl.pallas_call(
    _tc2_body,
    out_shape=jax.ShapeDtypeStruct((N, 128), jnp.float32),
)


def _tc3_body(accp_ref, dinv_ref, b2_ref, batch_ref, out_ref):
    h = ((accp_ref[0:N] + accp_ref[NP_:NP_ + N])
         * dinv_ref[...] + b2_ref[...])
    gids = lax.broadcasted_iota(jnp.int32, (1, G), 1)
    oh = (batch_ref[...] == gids).astype(jnp.float32)
    dn = (((0,), (0,)), ((), ()))
    sums = lax.dot_general(oh, h, dn, preferred_element_type=jnp.float32)
    counts = lax.dot_general(oh, jnp.ones((N, 1), jnp.float32), dn,
                             preferred_element_type=jnp.float32)
    out_ref[...] = sums / jnp.maximum(counts, 1.0)


_tc3 = pl.pallas_call(
    _tc3_body,
    out_shape=jax.ShapeDtypeStruct((G, 64), jnp.float32),
)


def kernel(x, edge_index, batch, edge_weight, W1, b1, W2, b2):
    x = x.astype(jnp.float32)
    ei = edge_index.astype(jnp.int32)
    ew = edge_weight.astype(jnp.float32)
    loop_ids = jnp.arange(N, dtype=jnp.int32)
    pad = E_PAD - ei.shape[1] - N
    zpad_i = jnp.zeros((pad,), jnp.int32)
    row = jnp.concatenate([ei[0], loop_ids, zpad_i])
    col = jnp.concatenate([ei[1], loop_ids, zpad_i])
    eww = jnp.concatenate([ew, jnp.ones((N,), jnp.float32),
                           jnp.zeros((pad,), jnp.float32)])
    row2 = row.reshape(E_PAD // PCH, PCH)
    col2 = col.reshape(E_PAD // PCH, PCH)
    ew2 = eww.reshape(E_PAD // PCH, PCH)

    degp = _deg(col, eww)
    hs1, dinv = _tc1(degp, x, W1)
    accp1 = _pass1(hs1, row2, col2, ew2)
    w2p = jnp.pad(W2, ((0, 0), (0, 64)))
    hs2 = _tc2(accp1, dinv, b1.reshape(1, 128), w2p)
    accp2 = _pass2(hs2, row2, col2, ew2)
    return _tc3(accp2, dinv, b2.reshape(1, 64),
                batch.astype(jnp.int32).reshape(N, 1))


# speedup vs baseline: 2.3579x; 2.3579x over previous
import jax
import jax.numpy as jnp
from jax import lax
from jax.experimental import pallas as pl
from jax.experimental.pallas import tpu as pltpu
from jax.experimental.pallas import tpu_sc as plsc

N = 10000
G = 16
NC = 2
NS = 16
NW = NC * NS
L = 16
CH = 128
PCH = 32
E_PAD = 331776
E_DEG = 344064
NP_ = 10240
RPT = NP_ // NS
ZR = 128


def _deg_body(col_hbm, ew_hbm, out_hbm, acc,
              cv0, cv1, cv2, cv3, ev0, ev1, ev2, st0, st1, st2, zrow,
              se0, se1, se2, sc0, sc1, sc2, sc3, ss0, ss1, ss2):
    cid = lax.axis_index("c")
    sid = lax.axis_index("s")
    e_per_tile = E_DEG // NW
    nchunks = e_per_tile // CH
    base = (cid * NS + sid) * e_per_tile
    zero = jnp.zeros((L,), jnp.float32)
    colv = (cv0, cv1, cv2, cv3)
    ewv = (ev0, ev1, ev2)
    stage = (st0, st1, st2)
    se = (se0, se1, se2)
    sc = (sc0, sc1, sc2, sc3)
    ss = (ss0, ss1, ss2)

    @pl.loop(0, ZR)
    def _(i):
        zrow[i, :] = zero

    @pl.loop(0, RPT // ZR)
    def _(k):
        pltpu.sync_copy(zrow, acc.at[pl.ds(sid * RPT + k * ZR, ZR)])
    plsc.subcore_barrier()

    for g in range(2):
        pltpu.sync_copy(ew_hbm.at[pl.ds(base + g * CH, CH)], ewv[g])
        pltpu.sync_copy(col_hbm.at[pl.ds(base + g * CH, CH)], colv[g])

    def chunk(g, k, kc):
        @pl.when(g >= 2)
        def _():
            pltpu.make_async_copy(stage[(k + 1) % 3], acc.at[colv[0]],
                                  ss[(k + 1) % 3]).wait()

        @pl.when(g + 2 < nchunks)
        def _():
            e2 = base + (g + 2) * CH
            pltpu.async_copy(ew_hbm.at[pl.ds(e2, CH)], ewv[(k + 2) % 3],
                             se[(k + 2) % 3])
            pltpu.async_copy(col_hbm.at[pl.ds(e2, CH)], colv[(kc + 2) % 4],
                             sc[(kc + 2) % 4])

        @pl.when(g >= 2)
        def _():
            pltpu.make_async_copy(ew_hbm.at[pl.ds(base, CH)], ewv[k],
                                  se[k]).wait()
            pltpu.make_async_copy(col_hbm.at[pl.ds(base, CH)], colv[kc],
                                  sc[kc]).wait()

        @pl.loop(0, CH // L)
        def _(kk):
            vew = ewv[k][pl.ds(kk * L, L)]
            for j in range(L):
                stage[k][kk * L + j, :] = jnp.broadcast_to(vew[j], (L,))

        pltpu.async_copy(stage[k], acc.at[colv[kc]], ss[k], add=True)

    nun = 12

    @pl.loop(0, nchunks // nun)
    def _(gg):
        g = gg * nun
        for u in range(nun):
            chunk(g + u, u % 3, u % 4)

    for gl in (nchunks - 2, nchunks - 1):
        pltpu.make_async_copy(stage[gl % 3], acc.at[colv[0]],
                              ss[gl % 3]).wait()

    plsc.subcore_barrier()
    pltpu.sync_copy(acc.at[pl.ds(sid * RPT, RPT)],
                    out_hbm.at[pl.ds(cid * NP_ + sid * RPT, RPT)])


def _make_pass_body(D, SC, DACC):
    nchunks = E_PAD // NW // PCH

    def body(hs_hbm, row_hbm, col_hbm, ew_hbm, out_hbm,
             acc, rv0, rv1, rv2, rv3, cv0, cv1, cv2, cv3,
             ev0, ev1, ev2, ev3, g0, g1, g2, g3, *rest):
        if DACC == D:
            (sr0, sr1, sr2, sr3, sc0, sc1, sc2, sc3,
             sg0, sg1, sg2, sg3, ss0, ss1, ss2, ss3) = rest
            cmpb = None
        else:
            (c0, c1, c2, c3,
             sr0, sr1, sr2, sr3, sc0, sc1, sc2, sc3,
             sg0, sg1, sg2, sg3, ss0, ss1, ss2, ss3) = rest
            cmpb = (c0, c1, c2, c3)
        cid = lax.axis_index("c")
        sid = lax.axis_index("s")
        e_per_tile = E_PAD // NW
        base = (cid * NS + sid) * e_per_tile
        zero = jnp.zeros((L,), jnp.float32)
        rowv = (rv0, rv1, rv2, rv3)
        colv = (cv0, cv1, cv2, cv3)
        ewv = (ev0, ev1, ev2, ev3)
        gath = (g0, g1, g2, g3)
        cmp_ = gath if DACC == D else cmpb
        sr = (sr0, sr1, sr2, sr3)
        sc = (sc0, sc1, sc2, sc3)
        sg = (sg0, sg1, sg2, sg3)
        ss = (ss0, ss1, ss2, ss3)

        @pl.loop(0, PCH)
        def _(i):
            for d in range(DACC // L):
                cmp_[0][i, pl.ds(d * L, L)] = zero

        @pl.loop(0, RPT // PCH)
        def _(k):
            pltpu.sync_copy(cmp_[0], acc.at[pl.ds(sid * RPT + k * PCH, PCH)])
        plsc.subcore_barrier()

        for g in range(3):
            e0 = base + g * PCH
            pltpu.sync_copy(row_hbm.at[pl.ds(e0, PCH)], rowv[g])
            pltpu.sync_copy(ew_hbm.at[pl.ds(e0, PCH)], ewv[g])
        for g in range(2):
            e0 = base + g * PCH
            pltpu.sync_copy(col_hbm.at[pl.ds(e0, PCH)], colv[g])
        pltpu.async_copy(hs_hbm.at[rowv[0]], g0, sg0)
        pltpu.async_copy(hs_hbm.at[rowv[1]], g1, sg1)

        def scale(k):
            @pl.loop(0, PCH // L)
            def _(kk):
                vew = ewv[k][pl.ds(kk * L, L)]
                for j in range(L):
                    sval = jnp.broadcast_to(vew[j], (L,))
                    r = kk * L + j
                    for d in range(SC // L):
                        cmp_[k][r, pl.ds(d * L, L)] = (
                            gath[k][r, pl.ds(d * L, L)] * sval)

        def chunk(g, k):
            k2 = (k + 2) % 4
            k3 = (k + 3) % 4
            @pl.when(g >= 2)
            def _():
                pltpu.make_async_copy(cmp_[k2], acc.at[colv[0]],
                                      ss[k2]).wait()

            @pl.when(g + 3 < nchunks)
            def _():
                e3 = base + (g + 3) * PCH
                pltpu.async_copy(row_hbm.at[pl.ds(e3, PCH)], rowv[k3], sr[k3])
                pltpu.async_copy(ew_hbm.at[pl.ds(e3, PCH)], ewv[k3], sr[k3])

            @pl.when(g + 2 < nchunks)
            def _():
                e2 = base + (g + 2) * PCH
                pltpu.async_copy(col_hbm.at[pl.ds(e2, PCH)], colv[k2], sc[k2])

                @pl.when(g >= 1)
                def _():
                    pltpu.make_async_copy(row_hbm.at[pl.ds(base, PCH)],
                                          rowv[k2], sr[k2]).wait()
                    pltpu.make_async_copy(ew_hbm.at[pl.ds(base, PCH)],
                                          rowv[k2], sr[k2]).wait()
                pltpu.async_copy(hs_hbm.at[rowv[k2]], gath[k2], sg[k2])

            pltpu.make_async_copy(hs_hbm.at[rowv[k]], gath[k], sg[k]).wait()
            scale(k)

            @pl.when(g >= 2)
            def _():
                pltpu.make_async_copy(col_hbm.at[pl.ds(base, PCH)],
                                      colv[k], sc[k]).wait()
            pltpu.async_copy(cmp_[k], acc.at[colv[k]], ss[k], add=True)

        @pl.loop(0, nchunks // 4)
        def _(gg):
            g = gg * 4
            for u in range(4):
                chunk(g + u, u)

        for gl in (nchunks - 2, nchunks - 1):
            pltpu.make_async_copy(cmp_[gl % 4], acc.at[colv[0]],
                                  ss[gl % 4]).wait()

        plsc.subcore_barrier()
        pltpu.sync_copy(acc.at[pl.ds(sid * RPT, RPT)],
                        out_hbm.at[pl.ds(cid * NP_ + sid * RPT, RPT)])

    return body


_mesh = plsc.VectorSubcoreMesh(core_axis_name="c", subcore_axis_name="s")

_deg = pl.kernel(
    _deg_body,
    out_type=jax.ShapeDtypeStruct((NC * NP_, L), jnp.float32),
    mesh=_mesh,
    scratch_types=(
        [pltpu.VMEM_SHARED((NP_, L), jnp.float32)]
        + [pltpu.VMEM((CH,), jnp.int32) for _ in range(4)]
        + [pltpu.VMEM((CH,), jnp.float32) for _ in range(3)]
        + [pltpu.VMEM((CH, L), jnp.float32) for _ in range(3)]
        + [pltpu.VMEM((ZR, L), jnp.float32)]
        + [pltpu.SemaphoreType.DMA for _ in range(10)]
    ),
)


def _make_pass(D, SC, DACC):
    extra = ([] if DACC == D
             else [pltpu.VMEM((PCH, DACC), jnp.float32) for _ in range(4)])
    return pl.kernel(
        _make_pass_body(D, SC, DACC),
        out_type=jax.ShapeDtypeStruct((NC * NP_, DACC), jnp.float32),
        mesh=_mesh,
        scratch_types=(
            [pltpu.VMEM_SHARED((NP_, DACC), jnp.float32)]
            + [pltpu.VMEM((PCH,), jnp.int32) for _ in range(8)]
            + [pltpu.VMEM((PCH,), jnp.float32) for _ in range(4)]
            + [pltpu.VMEM((PCH, D), jnp.float32) for _ in range(4)]
            + extra
            + [pltpu.SemaphoreType.DMA for _ in range(16)]
        ),
    )


_pass1 = _make_pass(128, 128, 128)
_pass2 = _make_pass(128, 64, 64)


def _tc1_body(deg_ref, x_ref, w1_ref, hs_ref, dinv_ref):
    deg = deg_ref[0:N, 0:1] + deg_ref[NP_:NP_ + N, 0:1]
    dinv = lax.rsqrt(deg)
    dinv_ref[...] = dinv
    hs_ref[...] = jnp.dot(x_ref[...], w1_ref[...],
                          preferred_element_type=jnp.float32) * dinv


_tc1 = pl.pallas_call(
    _tc1_body,
    out_shape=(jax.ShapeDtypeStruct((N, 128), jnp.float32),
               jax.ShapeDtypeStruct((N, 1), jnp.float32)),
)


def _tc2_body(accp_ref, dinv_ref, b1_ref, w2_ref, hs2_ref):
    dinv = dinv_ref[...]
    h = (accp_ref[0:N] + accp_ref[NP_:NP_ + N]) * dinv + b1_ref[...]
    h = jnp.maximum(h, 0.0)
    hs2_ref[...] = jnp.dot(h, w2_ref[...],
                           preferred_element_type=jnp.float32) * dinv


_tc2 = pl.pallas_call(
    _tc2_body,
    out_shape=jax.ShapeDtypeStruct((N, 128), jnp.float32),
)


def _tc3_body(accp_ref, dinv_ref, b2_ref, batch_ref, out_ref):
    h = ((accp_ref[0:N] + accp_ref[NP_:NP_ + N])
         * dinv_ref[...] + b2_ref[...])
    gids = lax.broadcasted_iota(jnp.int32, (1, G), 1)
    oh = (batch_ref[...] == gids).astype(jnp.float32)
    dn = (((0,), (0,)), ((), ()))
    sums = lax.dot_general(oh, h, dn, preferred_element_type=jnp.float32)
    counts = lax.dot_general(oh, jnp.ones((N, 1), jnp.float32), dn,
                             preferred_element_type=jnp.float32)
    out_ref[...] = sums / jnp.maximum(counts, 1.0)


_tc3 = pl.pallas_call(
    _tc3_body,
    out_shape=jax.ShapeDtypeStruct((G, 64), jnp.float32),
)


def kernel(x, edge_index, batch, edge_weight, W1, b1, W2, b2):
    x = x.astype(jnp.float32)
    ei = edge_index.astype(jnp.int32)
    ew = edge_weight.astype(jnp.float32)
    loop_ids = jnp.arange(N, dtype=jnp.int32)
    pad = E_PAD - ei.shape[1] - N
    zpad_i = jnp.zeros((pad,), jnp.int32)
    row = jnp.concatenate([ei[0], loop_ids, zpad_i])
    col = jnp.concatenate([ei[1], loop_ids, zpad_i])
    eww = jnp.concatenate([ew, jnp.ones((N,), jnp.float32),
                           jnp.zeros((pad,), jnp.float32)])
    dpad = E_DEG - E_PAD
    col_d = jnp.concatenate([col, jnp.zeros((dpad,), jnp.int32)])
    ew_d = jnp.concatenate([eww, jnp.zeros((dpad,), jnp.float32)])

    degp = _deg(col_d, ew_d)
    hs1, dinv = _tc1(degp, x, W1)
    accp1 = _pass1(hs1, row, col, eww)
    w2p = jnp.pad(W2, ((0, 0), (0, 64)))
    hs2 = _tc2(accp1, dinv, b1.reshape(1, 128), w2p)
    accp2 = _pass2(hs2, row, col, eww)
    return _tc3(accp2, dinv, b2.reshape(1, 64),
                batch.astype(jnp.int32).reshape(N, 1))


# speedup vs baseline: 2.5320x; 1.0738x over previous
import jax
import jax.numpy as jnp
from jax import lax
from jax.experimental import pallas as pl
from jax.experimental.pallas import tpu as pltpu
from jax.experimental.pallas import tpu_sc as plsc

N = 10000
G = 16
NC = 2
NS = 16
NW = NC * NS
L = 16
CH = 128
PCH = 32
E_PAD = 331776
E_DEG = 344064
NP_ = 10240
RPT = NP_ // NS
ZR = 128


def _deg_body(col_hbm, ew_hbm, out_hbm, acc,
              cv0, cv1, cv2, cv3, ev0, ev1, ev2, st0, st1, st2, zrow,
              se0, se1, se2, sc0, sc1, sc2, sc3, ss0, ss1, ss2):
    cid = lax.axis_index("c")
    sid = lax.axis_index("s")
    e_per_tile = E_DEG // NW
    nchunks = e_per_tile // CH
    base = (cid * NS + sid) * e_per_tile
    zero = jnp.zeros((L,), jnp.float32)
    colv = (cv0, cv1, cv2, cv3)
    ewv = (ev0, ev1, ev2)
    stage = (st0, st1, st2)
    se = (se0, se1, se2)
    sc = (sc0, sc1, sc2, sc3)
    ss = (ss0, ss1, ss2)

    @pl.loop(0, ZR)
    def _(i):
        zrow[i, :] = zero

    @pl.loop(0, RPT // ZR)
    def _(k):
        pltpu.sync_copy(zrow, acc.at[pl.ds(sid * RPT + k * ZR, ZR)])
    plsc.subcore_barrier()

    for g in range(2):
        pltpu.sync_copy(ew_hbm.at[pl.ds(base + g * CH, CH)], ewv[g])
        pltpu.sync_copy(col_hbm.at[pl.ds(base + g * CH, CH)], colv[g])

    def chunk(g, k, kc):
        @pl.when(g >= 2)
        def _():
            pltpu.make_async_copy(stage[(k + 1) % 3], acc.at[colv[0]],
                                  ss[(k + 1) % 3]).wait()

        @pl.when(g + 2 < nchunks)
        def _():
            e2 = base + (g + 2) * CH
            pltpu.async_copy(ew_hbm.at[pl.ds(e2, CH)], ewv[(k + 2) % 3],
                             se[(k + 2) % 3])
            pltpu.async_copy(col_hbm.at[pl.ds(e2, CH)], colv[(kc + 2) % 4],
                             sc[(kc + 2) % 4])

        @pl.when(g >= 2)
        def _():
            pltpu.make_async_copy(ew_hbm.at[pl.ds(base, CH)], ewv[k],
                                  se[k]).wait()
            pltpu.make_async_copy(col_hbm.at[pl.ds(base, CH)], colv[kc],
                                  sc[kc]).wait()

        @pl.loop(0, CH // L)
        def _(kk):
            vew = ewv[k][pl.ds(kk * L, L)]
            for j in range(L):
                stage[k][kk * L + j, :] = jnp.broadcast_to(vew[j], (L,))

        pltpu.async_copy(stage[k], acc.at[colv[kc]], ss[k], add=True)

    nun = 12

    @pl.loop(0, nchunks // nun)
    def _(gg):
        g = gg * nun
        for u in range(nun):
            chunk(g + u, u % 3, u % 4)

    for gl in (nchunks - 2, nchunks - 1):
        pltpu.make_async_copy(stage[gl % 3], acc.at[colv[0]],
                              ss[gl % 3]).wait()

    plsc.subcore_barrier()
    pltpu.sync_copy(acc.at[pl.ds(sid * RPT, RPT)],
                    out_hbm.at[pl.ds(cid * NP_ + sid * RPT, RPT)])


def _make_pass_body(D, SC, DACC):
    nchunks = E_PAD // NW // PCH

    def body(hs_hbm, row_hbm, col_hbm, ew_hbm, out_hbm,
             acc, rv0, rv1, rv2, rv3, cv0, cv1, cv2, cv3,
             ev0, ev1, ev2, ev3, g0, g1, g2, g3, *rest):
        if DACC == D:
            (sr0, sr1, sr2, sr3, sc0, sc1, sc2, sc3,
             sg0, sg1, sg2, sg3, ss0, ss1, ss2, ss3) = rest
            cmpb = None
        else:
            (c0, c1, c2, c3,
             sr0, sr1, sr2, sr3, sc0, sc1, sc2, sc3,
             sg0, sg1, sg2, sg3, ss0, ss1, ss2, ss3) = rest
            cmpb = (c0, c1, c2, c3)
        cid = lax.axis_index("c")
        sid = lax.axis_index("s")
        e_per_tile = E_PAD // NW
        base = (cid * NS + sid) * e_per_tile
        zero = jnp.zeros((L,), jnp.float32)
        rowv = (rv0, rv1, rv2, rv3)
        colv = (cv0, cv1, cv2, cv3)
        ewv = (ev0, ev1, ev2, ev3)
        gath = (g0, g1, g2, g3)
        cmp_ = gath if DACC == D else cmpb
        sr = (sr0, sr1, sr2, sr3)
        sc = (sc0, sc1, sc2, sc3)
        sg = (sg0, sg1, sg2, sg3)
        ss = (ss0, ss1, ss2, ss3)

        @pl.loop(0, PCH)
        def _(i):
            for d in range(DACC // L):
                cmp_[0][i, pl.ds(d * L, L)] = zero

        @pl.loop(0, RPT // PCH)
        def _(k):
            pltpu.sync_copy(cmp_[0], acc.at[pl.ds(sid * RPT + k * PCH, PCH)])
        plsc.subcore_barrier()

        for g in range(3):
            e0 = base + g * PCH
            pltpu.sync_copy(row_hbm.at[pl.ds(e0, PCH)], rowv[g])
            pltpu.sync_copy(ew_hbm.at[pl.ds(e0, PCH)], ewv[g])
        for g in range(2):
            e0 = base + g * PCH
            pltpu.sync_copy(col_hbm.at[pl.ds(e0, PCH)], colv[g])
        pltpu.async_copy(hs_hbm.at[rowv[0]], g0, sg0)
        pltpu.async_copy(hs_hbm.at[rowv[1]], g1, sg1)

        def scale(k):
            @pl.loop(0, PCH // L)
            def _(kk):
                vew = ewv[k][pl.ds(kk * L, L)]
                for j in range(L):
                    sval = jnp.broadcast_to(vew[j], (L,))
                    r = kk * L + j
                    for d in range(SC // L):
                        cmp_[k][r, pl.ds(d * L, L)] = (
                            gath[k][r, pl.ds(d * L, L)] * sval)

        def chunk(g, k):
            k2 = (k + 2) % 4
            k3 = (k + 3) % 4
            @pl.when(g >= 2)
            def _():
                pltpu.make_async_copy(cmp_[k2], acc.at[colv[0]],
                                      ss[k2]).wait()

            @pl.when(g + 3 < nchunks)
            def _():
                e3 = base + (g + 3) * PCH
                pltpu.async_copy(row_hbm.at[pl.ds(e3, PCH)], rowv[k3], sr[k3])
                pltpu.async_copy(ew_hbm.at[pl.ds(e3, PCH)], ewv[k3], sr[k3])

            @pl.when(g + 2 < nchunks)
            def _():
                e2 = base + (g + 2) * PCH
                pltpu.async_copy(col_hbm.at[pl.ds(e2, PCH)], colv[k2], sc[k2])

                @pl.when(g >= 1)
                def _():
                    pltpu.make_async_copy(row_hbm.at[pl.ds(base, PCH)],
                                          rowv[k2], sr[k2]).wait()
                    pltpu.make_async_copy(ew_hbm.at[pl.ds(base, PCH)],
                                          rowv[k2], sr[k2]).wait()
                pltpu.async_copy(hs_hbm.at[rowv[k2]], gath[k2], sg[k2])

            pltpu.make_async_copy(hs_hbm.at[rowv[k]], gath[k], sg[k]).wait()
            scale(k)

            @pl.when(g >= 2)
            def _():
                pltpu.make_async_copy(col_hbm.at[pl.ds(base, PCH)],
                                      colv[k], sc[k]).wait()
            pltpu.async_copy(cmp_[k], acc.at[colv[k]], ss[k], add=True)

        @pl.loop(0, nchunks // 4)
        def _(gg):
            g = gg * 4
            for u in range(4):
                chunk(g + u, u)

        for gl in (nchunks - 2, nchunks - 1):
            pltpu.make_async_copy(cmp_[gl % 4], acc.at[colv[0]],
                                  ss[gl % 4]).wait()

        plsc.subcore_barrier()
        pltpu.sync_copy(acc.at[pl.ds(sid * RPT, RPT)],
                        out_hbm.at[pl.ds(cid * NP_ + sid * RPT, RPT)])

    return body


_mesh = plsc.VectorSubcoreMesh(core_axis_name="c", subcore_axis_name="s")

_deg = pl.kernel(
    _deg_body,
    out_type=jax.ShapeDtypeStruct((NC * NP_, L), jnp.float32),
    mesh=_mesh,
    scratch_types=(
        [pltpu.VMEM_SHARED((NP_, L), jnp.float32)]
        + [pltpu.VMEM((CH,), jnp.int32) for _ in range(4)]
        + [pltpu.VMEM((CH,), jnp.float32) for _ in range(3)]
        + [pltpu.VMEM((CH, L), jnp.float32) for _ in range(3)]
        + [pltpu.VMEM((ZR, L), jnp.float32)]
        + [pltpu.SemaphoreType.DMA for _ in range(10)]
    ),
)


def _make_pass(D, SC, DACC):
    extra = ([] if DACC == D
             else [pltpu.VMEM((PCH, DACC), jnp.float32) for _ in range(4)])
    return pl.kernel(
        _make_pass_body(D, SC, DACC),
        out_type=jax.ShapeDtypeStruct((NC * NP_, DACC), jnp.float32),
        mesh=_mesh,
        scratch_types=(
            [pltpu.VMEM_SHARED((NP_, DACC), jnp.float32)]
            + [pltpu.VMEM((PCH,), jnp.int32) for _ in range(8)]
            + [pltpu.VMEM((PCH,), jnp.float32) for _ in range(4)]
            + [pltpu.VMEM((PCH, D), jnp.float32) for _ in range(4)]
            + extra
            + [pltpu.SemaphoreType.DMA for _ in range(16)]
        ),
    )


_pass1 = _make_pass(128, 128, 128)
_pass2 = _make_pass(128, 64, 64)


def _tc1_body(deg_ref, x_ref, w1_ref, hs_ref, dinv_ref):
    deg = deg_ref[0:N, 0:1] + deg_ref[NP_:NP_ + N, 0:1]
    dinv = lax.rsqrt(deg)
    dinv_ref[...] = dinv
    hs_ref[...] = jnp.dot(x_ref[...], w1_ref[...],
                          preferred_element_type=jnp.float32) * dinv


_tc1 = pl.pallas_call(
    _tc1_body,
    out_shape=(jax.ShapeDtypeStruct((N, 128), jnp.float32),
               jax.ShapeDtypeStruct((N, 1), jnp.float32)),
)


def _tc2_body(accp_ref, dinv_ref, b1_ref, w2_ref, hs2_ref):
    dinv = dinv_ref[...]
    h = (accp_ref[0:N] + accp_ref[NP_:NP_ + N]) * dinv + b1_ref[...]
    h = jnp.maximum(h, 0.0)
    hs2_ref[...] = jnp.dot(h, w2_ref[...],
                           preferred_element_type=jnp.float32) * dinv


_tc2 = pl.pallas_call(
    _tc2_body,
    out_shape=jax.ShapeDtypeStruct((N, 128), jnp.float32),
)


def _tc3_body(accp_ref, dinv_ref, b2_ref, batch_ref, out_ref):
    h = ((accp_ref[0:N] + accp_ref[NP_:NP_ + N])
         * dinv_ref[...] + b2_ref[...])
    gids = lax.broadcasted_iota(jnp.int32, (1, G), 1)
    oh = (batch_ref[...] == gids).astype(jnp.float32)
    dn = (((0,), (0,)), ((), ()))
    sums = lax.dot_general(oh, h, dn, preferred_element_type=jnp.float32)
    counts = lax.dot_general(oh, jnp.ones((N, 1), jnp.float32), dn,
                             preferred_element_type=jnp.float32)
    out_ref[...] = sums / jnp.maximum(counts, 1.0)


_tc3 = pl.pallas_call(
    _tc3_body,
    out_shape=jax.ShapeDtypeStruct((G, 64), jnp.float32),
)


def kernel(x, edge_index, batch, edge_weight, W1, b1, W2, b2):
    x = x.astype(jnp.float32)
    ei = edge_index.astype(jnp.int32)
    ew = edge_weight.astype(jnp.float32)
    loop_ids = jnp.arange(N, dtype=jnp.int32)
    pad = E_PAD - ei.shape[1] - N
    zpad_i = jnp.zeros((pad,), jnp.int32)
    row = jnp.concatenate([ei[0], loop_ids, zpad_i])
    col = jnp.concatenate([ei[1], loop_ids, zpad_i])
    eww = jnp.concatenate([ew, jnp.ones((N,), jnp.float32),
                           jnp.zeros((pad,), jnp.float32)])
    dpad = E_DEG - E_PAD
    col_d = jnp.concatenate([col, jnp.zeros((dpad,), jnp.int32)])
    ew_d = jnp.concatenate([eww, jnp.zeros((dpad,), jnp.float32)])

    def deal(a, ch):
        n = a.shape[0] // (NW * ch)
        return a.reshape(n, NW, ch).transpose(1, 0, 2).reshape(-1)

    row = deal(row, PCH)
    col = deal(col, PCH)
    eww = deal(eww, PCH)
    col_d = deal(col_d, CH)
    ew_d = deal(ew_d, CH)

    degp = _deg(col_d, ew_d)
    hs1, dinv = _tc1(degp, x, W1)
    accp1 = _pass1(hs1, row, col, eww)
    w2p = jnp.pad(W2, ((0, 0), (0, 64)))
    hs2 = _tc2(accp1, dinv, b1.reshape(1, 128), w2p)
    accp2 = _pass2(hs2, row, col, eww)
    return _tc3(accp2, dinv, b2.reshape(1, 64),
                batch.astype(jnp.int32).reshape(N, 1))
